# baseline (speedup 1.0000x reference)
import jax
import jax.numpy as jnp
import numpy as np
from jax.experimental import pallas as pl
from jax.experimental.pallas import tpu as pltpu

_H = 256
_W = 256
_N = 4096
_TH = 32
_TW = 32
_TP = _TH * _TW
_TN = 1024
_NTX = _W // _TW
_NTY = _H // _TH
_NTILES = _NTX * _NTY
_NCH = _N // _TN
_LOG2E = 1.4426950408889634


def _prep_kernel(xyz_ref, chol_ref, fdc_ref, rc_ref, op_ref,
                 params_ref, colors_ref):
    mean = jnp.tanh(xyz_ref[...])
    x = 0.5 * (mean[:, 0:1] + 1.0) * _W
    y = 0.5 * (mean[:, 1:2] + 1.0) * _H
    chol = chol_ref[...]
    l0 = chol[:, 0:1] + 0.5
    l1 = chol[:, 1:2]
    l2 = chol[:, 2:3] + 0.5
    s00 = l0 * l0
    s01 = l0 * l1
    s11 = l1 * l1 + l2 * l2
    det = s00 * s11 - s01 * s01
    f = -_LOG2E / det
    na = s11 * f
    nb = -s01 * f
    nc = s00 * f
    zeros = jnp.zeros_like(x)
    p = jnp.concatenate([x, y, 0.5 * na, 0.5 * nc, nb, na, nc, zeros], axis=1)
    params_ref[...] = p.T

    op = op_ref[...]
    ones = jnp.ones_like(x)
    cat = jnp.concatenate(
        [fdc_ref[...], 0.5 * rc_ref[...], ones, zeros], axis=1)
    colors_ref[...] = (cat * op).astype(jnp.bfloat16)


def _raster_kernel(basis_ref, params_ref, colors_ref, out_ref):
    t = pl.program_id(0)
    tx = t % _NTX
    ty = t // _NTX
    cx = (tx * _TW + _TW // 2).astype(jnp.float32)
    cy = (ty * _TH + _TH // 2).astype(jnp.float32)
    basis = basis_ref[...]
    acc = jnp.zeros((_TP, 8), dtype=jnp.float32)
    for k in range(_NCH):
        p = params_ref[:, k * _TN:(k + 1) * _TN]
        x = p[0:1, :]
        y = p[1:2, :]
        nA = p[2:3, :]
        nC = p[3:4, :]
        nb = p[4:5, :]
        na = p[5:6, :]
        nc = p[6:7, :]
        dx0 = cx - x
        dy0 = cy - y
        c0 = (nA * dx0 + nb * dy0) * dx0 + nC * dy0 * dy0
        c1 = na * dx0 + nb * dy0
        c2 = nc * dy0 + nb * dx0
        coeff = jnp.concatenate(
            [c0, c1, c2, nA, nC, nb, jnp.zeros((2, _TN), jnp.float32)], axis=0)
        sig = jax.lax.dot_general(
            basis, coeff, (((1,), (0,)), ((), ())),
            precision=jax.lax.Precision.HIGHEST,
            preferred_element_type=jnp.float32)
        w = jnp.exp2(sig).astype(jnp.bfloat16)
        col = colors_ref[k * _TN:(k + 1) * _TN, :]
        acc = acc + jax.lax.dot_general(
            w, col, (((1,), (0,)), ((), ())),
            preferred_element_type=jnp.float32)
    lane = jax.lax.broadcasted_iota(jnp.int32, (_TP, 8), 1)
    out_ref[...] = jnp.where(lane < 6, jnp.clip(acc, 0.0, 1.0), acc)[None]


def kernel(_xyz, _cholesky, _features_dc, random_colors, _opacity):
    params, colors = pl.pallas_call(
        _prep_kernel,
        out_shape=[
            jax.ShapeDtypeStruct((8, _N), jnp.float32),
            jax.ShapeDtypeStruct((_N, 8), jnp.bfloat16),
        ],
    )(_xyz, _cholesky, _features_dc, random_colors, _opacity)

    lx = np.arange(_TW, dtype=np.float32) + 0.5 - _TW // 2
    ly = np.arange(_TH, dtype=np.float32) + 0.5 - _TH // 2
    pxg, pyg = np.meshgrid(lx, ly)
    px = pxg.reshape(-1)
    py = pyg.reshape(-1)
    basis = np.stack(
        [np.ones_like(px), px, py, px * px, py * py, px * py,
         np.zeros_like(px), np.zeros_like(px)], axis=1)
    basis = jnp.asarray(basis, dtype=jnp.float32)

    acc = pl.pallas_call(
        _raster_kernel,
        grid=(_NTILES,),
        in_specs=[
            pl.BlockSpec((_TP, 8), lambda t: (0, 0)),
            pl.BlockSpec((8, _N), lambda t: (0, 0)),
            pl.BlockSpec((_N, 8), lambda t: (0, 0)),
        ],
        out_specs=pl.BlockSpec((1, _TP, 8), lambda t: (t, 0, 0)),
        out_shape=jax.ShapeDtypeStruct((_NTILES, _TP, 8), jnp.float32),
        compiler_params=pltpu.CompilerParams(
            dimension_semantics=("parallel",)),
    )(basis, params, colors)

    img = acc.reshape(_NTY, _NTX, _TH, _TW, 8).transpose(0, 2, 1, 3, 4)
    img = img.reshape(_H, _W, 8)
    out_img = img[:, :, 0:3].transpose(2, 0, 1)[None]
    gauss_img = img[:, :, 3:6].transpose(2, 0, 1)[None]
    out_alpha = img[:, :, 6:7].transpose(2, 0, 1)[None]
    return (out_img, gauss_img, out_alpha, _opacity)


# speedup vs baseline: 2.6326x; 2.6326x over previous
import jax
import jax.numpy as jnp
import numpy as np
from jax.experimental import pallas as pl
from jax.experimental.pallas import tpu as pltpu

_H = 256
_W = 256
_N = 4096
_TH = 32
_TW = 32
_TP = _TH * _TW
_TN = 1024
_NTX = _W // _TW
_NTY = _H // _TH
_NTILES = _NTX * _NTY
_NCH = _N // _TN
_LOG2E = 1.4426950408889634


def _prep_kernel(xyz_ref, chol_ref, fdc_ref, rc_ref, op_ref,
                 params_ref, colors_ref):
    mean = jnp.tanh(xyz_ref[...])
    x = 0.5 * (mean[:, 0:1] + 1.0) * _W
    y = 0.5 * (mean[:, 1:2] + 1.0) * _H
    chol = chol_ref[...]
    l0 = chol[:, 0:1] + 0.5
    l1 = chol[:, 1:2]
    l2 = chol[:, 2:3] + 0.5
    s00 = l0 * l0
    s01 = l0 * l1
    s11 = l1 * l1 + l2 * l2
    det = s00 * s11 - s01 * s01
    f = -_LOG2E / det
    na = s11 * f
    nb = -s01 * f
    nc = s00 * f
    zeros = jnp.zeros_like(x)
    p = jnp.concatenate([x, y, 0.5 * na, 0.5 * nc, nb, na, nc, zeros], axis=1)
    params_ref[...] = p.T

    op = op_ref[...]
    ones = jnp.ones_like(x)
    cat = jnp.concatenate(
        [fdc_ref[...], 0.5 * rc_ref[...], ones, zeros], axis=1)
    colors_ref[...] = (cat * op).astype(jnp.bfloat16)


def _hi_lo(v):
    hi = v.astype(jnp.bfloat16)
    lo = (v - hi.astype(jnp.float32)).astype(jnp.bfloat16)
    return hi, lo


def _raster_kernel(basis_ref, params_ref, colors_ref, out_ref):
    t = pl.program_id(0)
    tx = t % _NTX
    ty = t // _NTX
    cx = (tx * _TW + _TW // 2).astype(jnp.float32)
    cy = (ty * _TH + _TH // 2).astype(jnp.float32)
    basis = basis_ref[...]
    acc = jnp.zeros((_TP, 8), dtype=jnp.float32)
    for k in range(_NCH):
        p = params_ref[:, k * _TN:(k + 1) * _TN]
        x = p[0:1, :]
        y = p[1:2, :]
        nA = p[2:3, :]
        nC = p[3:4, :]
        nb = p[4:5, :]
        na = p[5:6, :]
        nc = p[6:7, :]
        dx0 = cx - x
        dy0 = cy - y
        c0 = (nA * dx0 + nb * dy0) * dx0 + nC * dy0 * dy0
        c1 = na * dx0 + nb * dy0
        c2 = nc * dy0 + nb * dx0
        c1h, c1l = _hi_lo(c1)
        c2h, c2l = _hi_lo(c2)
        c3h, c3l = _hi_lo(nA)
        c4h, c4l = _hi_lo(nC)
        c5h, c5l = _hi_lo(nb)
        zero = jnp.zeros((1, _TN), jnp.bfloat16)
        coeff = jnp.concatenate(
            [c1h, c1l, c2h, c2l,
             c3h, c3l, c3h,
             c4h, c4l, c4h,
             c5h, c5l, c5h,
             zero, zero, zero], axis=0)
        sig = jax.lax.dot_general(
            basis, coeff, (((1,), (0,)), ((), ())),
            preferred_element_type=jnp.float32)
        w = jnp.exp2(sig + c0).astype(jnp.bfloat16)
        col = colors_ref[k * _TN:(k + 1) * _TN, :]
        acc = acc + jax.lax.dot_general(
            w, col, (((1,), (0,)), ((), ())),
            preferred_element_type=jnp.float32)
    lane = jax.lax.broadcasted_iota(jnp.int32, (_TP, 8), 1)
    out_ref[...] = jnp.where(lane < 6, jnp.clip(acc, 0.0, 1.0), acc)[None]


def kernel(_xyz, _cholesky, _features_dc, random_colors, _opacity):
    params, colors = pl.pallas_call(
        _prep_kernel,
        out_shape=[
            jax.ShapeDtypeStruct((8, _N), jnp.float32),
            jax.ShapeDtypeStruct((_N, 8), jnp.bfloat16),
        ],
    )(_xyz, _cholesky, _features_dc, random_colors, _opacity)

    lx = np.arange(_TW, dtype=np.float32) + 0.5 - _TW // 2
    ly = np.arange(_TH, dtype=np.float32) + 0.5 - _TH // 2
    pxg, pyg = np.meshgrid(lx, ly)
    px = pxg.reshape(-1)
    py = pyg.reshape(-1)

    def split(v):
        hi = v.astype(jnp.bfloat16)
        lo = jnp.asarray(v) - hi.astype(jnp.float32)
        return hi.astype(np.float32), lo

    qxh, qxl = split(px * px)
    qyh, qyl = split(py * py)
    qxyh, qxyl = split(px * py)
    z = np.zeros_like(px)
    basis = jnp.stack(
        [px, px, py, py,
         qxh, qxh, qxl,
         qyh, qyh, qyl,
         qxyh, qxyh, qxyl,
         z, z, z], axis=1).astype(jnp.bfloat16)

    acc = pl.pallas_call(
        _raster_kernel,
        grid=(_NTILES,),
        in_specs=[
            pl.BlockSpec((_TP, 16), lambda t: (0, 0)),
            pl.BlockSpec((8, _N), lambda t: (0, 0)),
            pl.BlockSpec((_N, 8), lambda t: (0, 0)),
        ],
        out_specs=pl.BlockSpec((1, _TP, 8), lambda t: (t, 0, 0)),
        out_shape=jax.ShapeDtypeStruct((_NTILES, _TP, 8), jnp.float32),
        compiler_params=pltpu.CompilerParams(
            dimension_semantics=("parallel",)),
    )(basis, params, colors)

    img = acc.reshape(_NTY, _NTX, _TH, _TW, 8).transpose(0, 2, 1, 3, 4)
    img = img.reshape(_H, _W, 8)
    out_img = img[:, :, 0:3].transpose(2, 0, 1)[None]
    gauss_img = img[:, :, 3:6].transpose(2, 0, 1)[None]
    out_alpha = img[:, :, 6:7].transpose(2, 0, 1)[None]
    return (out_img, gauss_img, out_alpha, _opacity)
